# Initial kernel scaffold; baseline (speedup 1.0000x reference)
import jax
import jax.numpy as jnp
from jax.experimental import pallas as pl


def kernel(z_e, codebook):
    raise NotImplementedError("write your pallas kernel here")


# speedup vs baseline: 1.0027x; 1.0027x over previous
import functools

import jax
import jax.numpy as jnp
from jax import lax
from jax.experimental import pallas as pl
from jax.experimental.pallas import tpu as pltpu
from jax.experimental.pallas import tpu_sc as plsc

_N = 16384
_K = 8192
_D = 256
_TN = 256
_BETA = 0.25


def _vq_body(n_tiles, n_tokens, k, d, tn,
             z_ref, zz_ref, cb_ref, cc_ref,
             idx_ref, enc_ref, loss_ref, perp_ref,
             colsum_ref, lsum_ref):
    i = pl.program_id(0)
    z = z_ref[...]
    cb = cb_ref[...]
    e = lax.dot_general(z, cb, dimension_numbers=(((1,), (1,)), ((), ())),
                        preferred_element_type=jnp.float32)
    dis = (zz_ref[...] - 2.0 * e) + cc_ref[...]
    row_min = jnp.min(dis, axis=1, keepdims=True)
    iota = lax.broadcasted_iota(jnp.int32, (tn, k), 1)
    inf = jnp.float32(jnp.inf)
    seg_m, seg_i = [], []
    for lo, hi in ((0, 2736), (2736, 5472), (5472, k)):
        mask = (iota >= lo) & (iota < hi)
        dm = jnp.where(mask, dis, inf)
        m = jnp.min(dm, axis=1, keepdims=True)
        seg_m.append(m)
        seg_i.append(jnp.min(jnp.where(dm == m, iota, k), axis=1,
                             keepdims=True))
    acc_v = seg_m[0].astype(jnp.bfloat16).astype(jnp.float32)
    acc_i = seg_i[0]
    t2 = seg_m[1] < acc_v
    acc_v = jnp.where(t2, seg_m[1], acc_v).astype(jnp.bfloat16).astype(jnp.float32)
    acc_i = jnp.where(t2, seg_i[1], acc_i)
    amin = jnp.where(seg_m[2] < acc_v, seg_i[2], acc_i)
    idx_ref[...] = amin
    enc = (iota == amin).astype(jnp.int32)
    enc_ref[...] = enc
    colsum = jnp.sum(enc, axis=0, keepdims=True)
    tile_loss = jnp.sum(row_min)

    @pl.when(i == 0)
    def _():
        colsum_ref[...] = colsum
        lsum_ref[0, 0] = tile_loss

    @pl.when(i > 0)
    def _():
        colsum_ref[...] += colsum
        lsum_ref[0, 0] += tile_loss

    @pl.when(i == n_tiles - 1)
    def _():
        e_mean = colsum_ref[...].astype(jnp.float32) / n_tokens
        ent = jnp.sum(e_mean * jnp.log(e_mean + 1e-10))
        perp_ref[0, 0] = jnp.exp(-ent)
        loss_ref[0, 0] = _BETA * lsum_ref[0, 0] / (n_tokens * d)


def _make_distance_kernel(n_tokens, k, d, tn, interpret=False):
    n_tiles = n_tokens // tn
    body = functools.partial(_vq_body, n_tiles, n_tokens, k, d, tn)
    return pl.pallas_call(
        body,
        grid=(n_tiles,),
        in_specs=[
            pl.BlockSpec((tn, d), lambda i: (i, 0)),
            pl.BlockSpec((tn, 1), lambda i: (i, 0)),
            pl.BlockSpec((k, d), lambda i: (0, 0)),
            pl.BlockSpec((1, k), lambda i: (0, 0)),
        ],
        out_specs=[
            pl.BlockSpec((tn, 1), lambda i: (i, 0)),
            pl.BlockSpec((tn, k), lambda i: (i, 0)),
            pl.BlockSpec(memory_space=pltpu.SMEM),
            pl.BlockSpec(memory_space=pltpu.SMEM),
        ],
        out_shape=[
            jax.ShapeDtypeStruct((n_tokens, 1), jnp.int32),
            jax.ShapeDtypeStruct((n_tokens, k), jnp.int32),
            jax.ShapeDtypeStruct((1, 1), jnp.float32),
            jax.ShapeDtypeStruct((1, 1), jnp.float32),
        ],
        scratch_shapes=[
            pltpu.VMEM((1, k), jnp.int32),
            pltpu.SMEM((1, 1), jnp.float32),
        ],
        compiler_params=pltpu.CompilerParams(
            dimension_semantics=("arbitrary",)),
        interpret=interpret,
    )


def _make_sc_gather(n_rows, d):
    info = plsc.get_sparse_core_info()
    nw = info.num_cores * info.num_subcores
    rows_per_w = n_rows // nw
    chunk = 128
    n_chunks = rows_per_w // chunk
    mesh = plsc.VectorSubcoreMesh(core_axis_name="c", subcore_axis_name="s")

    @functools.partial(
        pl.kernel, mesh=mesh,
        out_type=jax.ShapeDtypeStruct((n_rows, d), jnp.float32),
        scratch_types=[
            pltpu.VMEM((chunk,), jnp.int32),
            pltpu.VMEM((chunk, d), jnp.float32),
            pltpu.SemaphoreType.DMA,
        ],
    )
    def gather(cb_hbm, idx_hbm, out_hbm, idx_v, rows_v, sem):
        wid = lax.axis_index("s") * info.num_cores + lax.axis_index("c")
        for c in range(n_chunks):
            base = wid * rows_per_w + c * chunk
            pltpu.sync_copy(idx_hbm.at[pl.ds(base, chunk)], idx_v)
            pltpu.async_copy(cb_hbm.at[idx_v], rows_v, sem).wait()
            pltpu.sync_copy(rows_v, out_hbm.at[pl.ds(base, chunk)])

    return gather


def kernel(z_e, codebook):
    b, c, h, w = z_e.shape
    z_flat = jnp.transpose(z_e, (0, 2, 3, 1)).reshape(-1, _D)
    zz = jnp.sum(z_flat ** 2, axis=1, keepdims=True)
    cc = jnp.sum(codebook ** 2, axis=1, keepdims=True).T
    idx2, enc, loss, perp = _make_distance_kernel(_N, _K, _D, _TN)(
        z_flat, zz, codebook, cc)
    z_q = _make_sc_gather(_N, _D)(codebook, idx2.reshape(-1))
    z_q_out = jnp.transpose(z_q.reshape(b, h, w, c), (0, 3, 1, 2))
    return (loss[0, 0], z_q_out, perp[0, 0], enc, idx2)
